# baseline (speedup 1.0000x reference)
import jax
import jax.numpy as jnp
from jax import lax
from jax.experimental import pallas as pl
from jax.experimental.pallas import tpu as pltpu
from jax.experimental.pallas import tpu_sc as plsc

VOCAB = 1_000_000
EMBED = 64
BATCH = 4096
HIST = 200

_NC = 2
_NS = 16
_NW = _NC * _NS
_B = BATCH * HIST
_CHUNK = 128
_CHUNKS = _B // (_NW * _CHUNK)
_NSLOT = 4
_CPS = 2
_GRP = _NSLOT * _CPS
_GROUPS = _CHUNKS // _GRP


def _gather_body(w_hbm, data_hbm, out_hbm, idx_v, bufs_v, gsem, osem):
    wid = lax.axis_index("s") * _NC + lax.axis_index("c")
    pltpu.sync_copy(data_hbm.at[wid], idx_v)
    row_base = wid * _CHUNKS * _CHUNK

    def fire_gather(c, s, h):
        pltpu.async_copy(w_hbm.at[idx_v.at[c]],
                         bufs_v.at[s, pl.ds(h * _CHUNK, _CHUNK)], gsem.at[s])

    def wait_gather(c, s, h):
        pltpu.make_async_copy(
            w_hbm.at[idx_v.at[c]],
            bufs_v.at[s, pl.ds(h * _CHUNK, _CHUNK)], gsem.at[s]).wait()

    def out_slice(c0):
        return out_hbm.at[pl.ds(row_base + c0 * _CHUNK, _CPS * _CHUNK)]

    def fire_copyout(c0, s):
        pltpu.async_copy(bufs_v.at[s], out_slice(c0), osem.at[s])

    def wait_copyout(c0, s):
        pltpu.make_async_copy(bufs_v.at[s], out_slice(c0), osem.at[s]).wait()

    for s in range(_NSLOT):
        for h in range(_CPS):
            fire_gather(s * _CPS + h, s, h)

    def group_step(g, carry):
        for s in range(_NSLOT):
            c0 = g * _GRP + s * _CPS
            for h in range(_CPS):
                wait_gather(c0 + h, s, h)
            fire_copyout(c0, s)
        for s in range(_NSLOT):
            c0 = g * _GRP + s * _CPS
            wait_copyout(c0, s)
            for h in range(_CPS):
                fire_gather(c0 + _GRP + h, s, h)
        return carry

    lax.fori_loop(0, _GROUPS - 1, group_step, 0)

    for s in range(_NSLOT):
        c0 = (_GROUPS - 1) * _GRP + s * _CPS
        for h in range(_CPS):
            wait_gather(c0 + h, s, h)
        fire_copyout(c0, s)
    for s in range(_NSLOT):
        c0 = (_GROUPS - 1) * _GRP + s * _CPS
        wait_copyout(c0, s)


def kernel(data, W):
    idx = data.reshape(_NW, _CHUNKS, _CHUNK)
    mesh = plsc.VectorSubcoreMesh(core_axis_name="c", subcore_axis_name="s")
    out_flat = pl.kernel(
        _gather_body,
        mesh=mesh,
        compiler_params=pltpu.CompilerParams(use_tc_tiling_on_sc=False),
        out_type=jax.ShapeDtypeStruct((_B, EMBED), jnp.float32),
        scratch_types=[
            pltpu.VMEM((_CHUNKS, _CHUNK), jnp.int32),
            pltpu.VMEM((_NSLOT, _CPS * _CHUNK, EMBED), jnp.float32),
            pltpu.SemaphoreType.DMA((_NSLOT,)),
            pltpu.SemaphoreType.DMA((_NSLOT,)),
        ],
    )(W, idx)
    return out_flat.reshape(BATCH, HIST, EMBED)
